# baseline (speedup 1.0000x reference)
import functools

import jax
import jax.numpy as jnp
from jax import lax
from jax.experimental import pallas as pl
from jax.experimental.pallas import tpu as pltpu
from jax.experimental.pallas import tpu_sc as plsc

B = 16384
EMBED = 64
NC = 2
NS = 16
NW = NC * NS
BPW = B // NW
LANES = 16
UNROLL = 4

_mesh = plsc.VectorSubcoreMesh(core_axis_name="c", subcore_axis_name="s")


@functools.partial(
    pl.kernel,
    out_type=(
        jax.ShapeDtypeStruct((NW, LANES), jnp.float32),
        jax.ShapeDtypeStruct((B,), jnp.float32),
    ),
    mesh=_mesh,
    scratch_types=[
        pltpu.VMEM((BPW,), jnp.int32),
        pltpu.VMEM((BPW,), jnp.int32),
        pltpu.VMEM((BPW // 4, 2 * EMBED), jnp.float32),
        pltpu.VMEM((BPW // 4, 2 * EMBED), jnp.float32),
        pltpu.VMEM((BPW // 4, 2 * EMBED), jnp.float32),
        pltpu.VMEM((BPW // 4, 2 * EMBED), jnp.float32),
        pltpu.VMEM((BPW,), jnp.int32),
        pltpu.VMEM((BPW,), jnp.int32),
        pltpu.VMEM((BPW,), jnp.float32),
        pltpu.VMEM((BPW,), jnp.float32),
        pltpu.VMEM((BPW,), jnp.float32),
        pltpu.VMEM((LANES,), jnp.float32),
        pltpu.SemaphoreType.DMA,
        pltpu.SemaphoreType.DMA,
    ],
)
def _gather_partials(idx_u_hbm, idx_b_hbm, uemb_hbm, bemb_hbm, ubias_hbm,
                     bbias_hbm, part_out, bsum_out,
                     idx_u_v, idx_b_v, urows, brows, urows2, brows2,
                     hoff_u, hoff_b, ubias_v, bbias_v, bsum_v, acc_v, sem,
                     rsem):
    wid = lax.axis_index("s") * NC + lax.axis_index("c")
    base = wid * BPW

    pltpu.sync_copy(idx_u_hbm.at[pl.ds(base, BPW)], idx_u_v)
    pltpu.sync_copy(idx_b_hbm.at[pl.ds(base, BPW)], idx_b_v)

    cub = pltpu.async_copy(ubias_hbm.at[idx_u_v], ubias_v, sem)
    cbb = pltpu.async_copy(bbias_hbm.at[idx_b_v], bbias_v, sem)

    def half_body(i, carry):
        sl = pl.ds(pl.multiple_of(i * LANES, LANES), LANES)
        hoff_u[sl] = (idx_u_v[sl] & 1) * EMBED
        hoff_b[sl] = (idx_b_v[sl] & 1) * EMBED
        return carry

    lax.fori_loop(0, BPW // LANES, half_body, 0)


    CHUNK = BPW // 4
    NCHUNK = 4

    def issue_chunk(c, urows, brows):
        def issue_body(w, carry):
            rbase = w * LANES
            vu = idx_u_v[pl.ds(c * CHUNK + rbase, LANES)]
            vb = idx_b_v[pl.ds(c * CHUNK + rbase, LANES)]
            for k in range(LANES):
                pltpu.async_copy(uemb_hbm.at[lax.shift_right_logical(
                    vu[k], 1)], urows.at[rbase + k], rsem)
                pltpu.async_copy(bemb_hbm.at[lax.shift_right_logical(
                    vb[k], 1)], brows.at[rbase + k], rsem)
            return carry

        lax.fori_loop(0, CHUNK // LANES, issue_body, 0)

    def drain_chunk(urows, brows):
        pltpu.make_async_copy(uemb_hbm.at[pl.ds(0, CHUNK), :], urows,
                              rsem).wait()
        pltpu.make_async_copy(bemb_hbm.at[pl.ds(0, CHUNK), :], brows,
                              rsem).wait()

    def compute_chunk(acc, c, urows, brows):
        def row_body(w, acc):
            hu = hoff_u[pl.ds(c * CHUNK + w * LANES, LANES)]
            hb = hoff_b[pl.ds(c * CHUNK + w * LANES, LANES)]
            for k in range(LANES):
                r = w * LANES + k
                for j in range(EMBED // LANES):
                    su = pl.ds(hu[k] + j * LANES, LANES)
                    sb = pl.ds(hb[k] + j * LANES, LANES)
                    acc = acc + urows[r, su] * brows[r, sb]
            return acc

        return lax.fori_loop(0, CHUNK // LANES, row_body, acc)

    bufs = [(urows, brows), (urows2, brows2)]
    acc = jnp.zeros((LANES,), jnp.float32)
    issue_chunk(0, *bufs[0])
    for c in range(NCHUNK):
        cur = bufs[c % 2]
        drain_chunk(*cur)
        if c + 1 < NCHUNK:
            issue_chunk(c + 1, *bufs[(c + 1) % 2])
        acc = compute_chunk(acc, c, *cur)
    acc_v[...] = acc
    pltpu.sync_copy(acc_v, part_out.at[wid])

    cub.wait()
    cbb.wait()

    def bias_body(i, carry):
        s = pl.ds(pl.multiple_of(i * LANES, LANES), LANES)
        bsum_v[s] = ubias_v[s] + bbias_v[s]
        return carry

    lax.fori_loop(0, BPW // LANES, bias_body, 0)
    pltpu.sync_copy(bsum_v, bsum_out.at[pl.ds(base, BPW)])


def _finalize_body(p_ref, b_ref, o_ref):
    s = jnp.sum(p_ref[...])
    x = b_ref[...] + s
    o_ref[...] = 1.0 / (1.0 + jnp.exp(-x))


_finalize = pl.pallas_call(
    _finalize_body,
    out_shape=jax.ShapeDtypeStruct((128, 128), jnp.float32),
)


def kernel(inputs, user_embedding, user_bias, book_embedding, book_bias):
    idx_u = inputs[:, 0].astype(jnp.int32)
    idx_b = inputs[:, 1].astype(jnp.int32)
    ub_flat = user_bias.reshape(-1)
    bb_flat = book_bias.reshape(-1)
    uemb2 = user_embedding.reshape(-1, 2 * EMBED)
    bemb2 = book_embedding.reshape(-1, 2 * EMBED)
    partials, bsum = _gather_partials(idx_u, idx_b, uemb2, bemb2, ub_flat,
                                      bb_flat)
    out = _finalize(partials, bsum.reshape(128, 128))
    return out.reshape(B, 1)


# speedup vs baseline: 1.3817x; 1.3817x over previous
import functools

import jax
import jax.numpy as jnp
from jax import lax
from jax.experimental import pallas as pl
from jax.experimental.pallas import tpu as pltpu
from jax.experimental.pallas import tpu_sc as plsc

B = 16384
EMBED = 64
NC = 2
NS = 16
NW = NC * NS
BPW = B // NW
LANES = 16
UNROLL = 4

_mesh = plsc.VectorSubcoreMesh(core_axis_name="c", subcore_axis_name="s")


@functools.partial(
    pl.kernel,
    out_type=(
        jax.ShapeDtypeStruct((NW, LANES), jnp.float32),
        jax.ShapeDtypeStruct((B,), jnp.float32),
    ),
    mesh=_mesh,
    scratch_types=[
        pltpu.VMEM((BPW,), jnp.int32),
        pltpu.VMEM((BPW,), jnp.int32),
        pltpu.VMEM((BPW // 4, EMBED), jnp.float32),
        pltpu.VMEM((BPW // 4, EMBED), jnp.float32),
        pltpu.VMEM((BPW // 4, EMBED), jnp.float32),
        pltpu.VMEM((BPW // 4, EMBED), jnp.float32),
        pltpu.VMEM((BPW,), jnp.float32),
        pltpu.VMEM((BPW,), jnp.float32),
        pltpu.VMEM((BPW,), jnp.float32),
        pltpu.VMEM((LANES,), jnp.float32),
        pltpu.SemaphoreType.DMA,
        pltpu.SemaphoreType.DMA,
    ],
)
def _gather_partials(idx_u_hbm, idx_b_hbm, uemb_hbm, bemb_hbm, ubias_hbm,
                     bbias_hbm, part_out, bsum_out,
                     idx_u_v, idx_b_v, urows, brows, urows2, brows2,
                     ubias_v, bbias_v, bsum_v, acc_v, sem, rsem):
    wid = lax.axis_index("s") * NC + lax.axis_index("c")
    base = wid * BPW

    pltpu.sync_copy(idx_u_hbm.at[pl.ds(base, BPW)], idx_u_v)
    pltpu.sync_copy(idx_b_hbm.at[pl.ds(base, BPW)], idx_b_v)

    cub = pltpu.async_copy(ubias_hbm.at[idx_u_v], ubias_v, sem)
    cbb = pltpu.async_copy(bbias_hbm.at[idx_b_v], bbias_v, sem)


    CHUNK = BPW // 4
    NCHUNK = 4

    def issue_chunk(c, urows, brows):
        def issue_body(w, carry):
            rbase = w * LANES
            vu = idx_u_v[pl.ds(c * CHUNK + rbase, LANES)]
            vb = idx_b_v[pl.ds(c * CHUNK + rbase, LANES)]
            for k in range(LANES):
                pltpu.async_copy(uemb_hbm.at[vu[k]], urows.at[rbase + k],
                                 rsem)
                pltpu.async_copy(bemb_hbm.at[vb[k]], brows.at[rbase + k],
                                 rsem)
            return carry

        lax.fori_loop(0, CHUNK // LANES, issue_body, 0)

    def drain_chunk(urows, brows):
        pltpu.make_async_copy(uemb_hbm.at[pl.ds(0, CHUNK), :], urows,
                              rsem).wait()
        pltpu.make_async_copy(bemb_hbm.at[pl.ds(0, CHUNK), :], brows,
                              rsem).wait()

    def compute_chunk(acc, urows, brows):
        def row_body(r, acc):
            for j in range(EMBED // LANES):
                s = pl.ds(j * LANES, LANES)
                acc = acc + urows[r, s] * brows[r, s]
            return acc

        return lax.fori_loop(0, CHUNK, row_body, acc)

    bufs = [(urows, brows), (urows2, brows2)]
    acc = jnp.zeros((LANES,), jnp.float32)
    issue_chunk(0, *bufs[0])
    for c in range(NCHUNK):
        cur = bufs[c % 2]
        drain_chunk(*cur)
        if c + 1 < NCHUNK:
            issue_chunk(c + 1, *bufs[(c + 1) % 2])
        acc = compute_chunk(acc, *cur)
    acc_v[...] = acc
    pltpu.sync_copy(acc_v, part_out.at[wid])

    cub.wait()
    cbb.wait()

    def bias_body(i, carry):
        s = pl.ds(pl.multiple_of(i * LANES, LANES), LANES)
        bsum_v[s] = ubias_v[s] + bbias_v[s]
        return carry

    lax.fori_loop(0, BPW // LANES, bias_body, 0)
    pltpu.sync_copy(bsum_v, bsum_out.at[pl.ds(base, BPW)])


def _finalize_body(p_ref, b_ref, o_ref):
    s = jnp.sum(p_ref[...])
    x = b_ref[...] + s
    o_ref[...] = 1.0 / (1.0 + jnp.exp(-x))


_finalize = pl.pallas_call(
    _finalize_body,
    out_shape=jax.ShapeDtypeStruct((128, 128), jnp.float32),
)


def kernel(inputs, user_embedding, user_bias, book_embedding, book_bias):
    idx_u = inputs[:, 0].astype(jnp.int32)
    idx_b = inputs[:, 1].astype(jnp.int32)
    ub_flat = user_bias.reshape(-1)
    bb_flat = book_bias.reshape(-1)
    partials, bsum = _gather_partials(idx_u, idx_b, user_embedding,
                                      book_embedding, ub_flat, bb_flat)
    out = _finalize(partials, bsum.reshape(128, 128))
    return out.reshape(B, 1)
